# baseline (speedup 1.0000x reference)
import functools

import jax
import jax.numpy as jnp
from jax import lax
from jax.experimental import pallas as pl
from jax.experimental.pallas import tpu as pltpu
from jax.experimental.pallas import tpu_sc as plsc

E = 8
CAP = 512
EC = E * CAP
H = 2048
FF = 8192
T = 8192


_RBT = 512


def _router_kernel(x_ref, wr_ref, p0_ref, p1_ref, i0_ref, i1_ref):
    x = x_ref[...]
    wr = wr_ref[...]
    logits = lax.dot_general(x, wr, (((1,), (1,)), ((), ())),
                             preferred_element_type=jnp.float32)
    m = jnp.max(logits, axis=-1, keepdims=True)
    ex = jnp.exp(logits - m)
    probs = ex / jnp.sum(ex, axis=-1, keepdims=True)
    e_iota = lax.broadcasted_iota(jnp.int32, probs.shape, 1)
    big = jnp.int32(999)
    p0 = jnp.max(probs, axis=-1, keepdims=True)
    i0 = jnp.min(jnp.where(probs == p0, e_iota, big), axis=-1, keepdims=True)
    masked = jnp.where(e_iota == i0, jnp.float32(-1.0), probs)
    p1 = jnp.max(masked, axis=-1, keepdims=True)
    i1 = jnp.min(jnp.where(masked == p1, e_iota, big), axis=-1, keepdims=True)
    p0_ref[...] = p0
    p1_ref[...] = p1
    i0_ref[...] = i0
    i1_ref[...] = i1


def _run_router(x, wr):
    grid = (T // _RBT,)
    return pl.pallas_call(
        _router_kernel,
        grid=grid,
        in_specs=[
            pl.BlockSpec((_RBT, H), lambda i: (i, 0)),
            pl.BlockSpec((E, H), lambda i: (0, 0)),
        ],
        out_specs=[
            pl.BlockSpec((_RBT, 1), lambda i: (i, 0)),
            pl.BlockSpec((_RBT, 1), lambda i: (i, 0)),
            pl.BlockSpec((_RBT, 1), lambda i: (i, 0)),
            pl.BlockSpec((_RBT, 1), lambda i: (i, 0)),
        ],
        out_shape=[
            jax.ShapeDtypeStruct((T, 1), jnp.float32),
            jax.ShapeDtypeStruct((T, 1), jnp.float32),
            jax.ShapeDtypeStruct((T, 1), jnp.int32),
            jax.ShapeDtypeStruct((T, 1), jnp.int32),
        ],
    )(x, wr)


def _meta_kernel(i0_ref, i1_ref, p0_ref, p1_ref,
                 ds0_ref, ds1_ref, d0_ref, d1_ref, w0_ref, w1_ref):
    e0 = i0_ref[...]
    e1 = i1_ref[...]
    p0 = p0_ref[...]
    p1 = p1_ref[...]

    r_iota = lax.broadcasted_iota(jnp.int32, (128, 128), 0)
    c_iota = lax.broadcasted_iota(jnp.int32, (128, 128), 1)
    U128 = jnp.where(r_iota < c_iota, 1.0, 0.0).astype(jnp.float32)
    r64 = lax.broadcasted_iota(jnp.int32, (64, 64), 0)
    c64 = lax.broadcasted_iota(jnp.int32, (64, 64), 1)
    L64 = jnp.where(c64 < r64, 1.0, 0.0).astype(jnp.float32)
    ones_col = jnp.ones((128, 1), jnp.float32)

    rank0 = jnp.zeros((64, 128), jnp.float32)
    rank1 = jnp.zeros((64, 128), jnp.float32)
    for e in range(E):
        a = (e0 == e).astype(jnp.float32)
        b = (e1 == e).astype(jnp.float32)
        s = a + b
        pfx = lax.dot_general(s, U128, (((1,), (0,)), ((), ())),
                              preferred_element_type=jnp.float32)
        rs = lax.dot_general(s, ones_col, (((1,), (0,)), ((), ())),
                             preferred_element_type=jnp.float32)
        offs = lax.dot_general(L64, rs, (((1,), (0,)), ((), ())),
                               preferred_element_type=jnp.float32)
        excl = pfx + offs
        rank0 = rank0 + a * excl
        rank1 = rank1 + b * (excl + a)

    e0f = e0.astype(jnp.float32)
    e1f = e1.astype(jnp.float32)
    dest0 = e0f * float(CAP) + rank0
    dest1 = e1f * float(CAP) + rank1
    v0 = rank0 < float(CAP)
    v1 = rank1 < float(CAP)
    dest0i = dest0.astype(jnp.int32)
    dest1i = dest1.astype(jnp.int32)
    ds0_ref[...] = jnp.where(v0, dest0i, EC)
    ds1_ref[...] = jnp.where(v1, dest1i, EC)
    d0_ref[...] = jnp.where(v0, dest0i, 0)
    d1_ref[...] = jnp.where(v1, dest1i, 0)
    w0_ref[...] = p0 * v0.astype(jnp.float32)
    w1_ref[...] = p1 * v1.astype(jnp.float32)


def _run_meta(i0, i1, p0, p1):
    spec = pl.BlockSpec((64, 128), lambda: (0, 0))
    return pl.pallas_call(
        _meta_kernel,
        in_specs=[spec] * 4,
        out_specs=[spec] * 6,
        out_shape=[
            jax.ShapeDtypeStruct((64, 128), jnp.int32),
            jax.ShapeDtypeStruct((64, 128), jnp.int32),
            jax.ShapeDtypeStruct((64, 128), jnp.int32),
            jax.ShapeDtypeStruct((64, 128), jnp.int32),
            jax.ShapeDtypeStruct((64, 128), jnp.float32),
            jax.ShapeDtypeStruct((64, 128), jnp.float32),
        ],
    )(i0, i1, p0, p1)


PEC = EC + 8


def _sc_wid():
    return lax.axis_index("s") * 2 + lax.axis_index("c")


def _make_sc_dispatch(mesh, chunk):
    n_workers = 32
    per_w = T // n_workers
    n_chunks = per_w // chunk

    @functools.partial(
        pl.kernel,
        mesh=mesh,
        out_type=jax.ShapeDtypeStruct((PEC, H // 2), jnp.float32),
        scratch_types=[
            pltpu.VMEM((chunk,), jnp.int32),
            pltpu.VMEM((chunk,), jnp.int32),
            pltpu.VMEM((chunk,), jnp.int32),
            pltpu.VMEM((chunk,), jnp.int32),
            pltpu.VMEM((chunk, H // 2), jnp.float32),
            pltpu.VMEM((chunk, H // 2), jnp.float32),
            pltpu.SemaphoreType.DMA,
            pltpu.SemaphoreType.DMA,
            pltpu.SemaphoreType.DMA,
            pltpu.SemaphoreType.DMA,
        ],
    )
    def dispatch(x_hbm, ds0_hbm, ds1_hbm, out_hbm,
                 i0a, i0b, i1a, i1b, ra, rb, sia, sib, soa, sob):
        wid = _sc_wid()
        i0v = (i0a, i0b)
        i1v = (i1a, i1b)
        rv = (ra, rb)
        si = (sia, sib)
        so = (soa, sob)

        def start_in(c):
            p = c % 2
            base = pl.multiple_of(wid * per_w + c * chunk, chunk)
            pltpu.sync_copy(ds0_hbm.at[pl.ds(base, chunk)], i0v[p])
            pltpu.sync_copy(ds1_hbm.at[pl.ds(base, chunk)], i1v[p])
            return pltpu.async_copy(x_hbm.at[pl.ds(base, chunk)], rv[p], si[p])

        pend_in = [start_in(0), None]
        pend_out = [None, None]
        for c in range(n_chunks):
            p = c % 2
            q = (c + 1) % 2
            pend_in[p].wait()
            d0 = pltpu.async_copy(rv[p], out_hbm.at[i0v[p]], so[p])
            d1 = pltpu.async_copy(rv[p], out_hbm.at[i1v[p]], so[p])
            if c + 1 < n_chunks:
                if pend_out[q] is not None:
                    for d in pend_out[q]:
                        d.wait()
                    pend_out[q] = None
                pend_in[q] = start_in(c + 1)
            pend_out[p] = (d0, d1)
        for pr in pend_out:
            if pr is not None:
                for d in pr:
                    d.wait()

    return dispatch


def _make_sc_gather(mesh, n_rows, chunk):
    n_workers = 32
    per_w = n_rows // n_workers
    n_chunks = per_w // chunk

    @functools.partial(
        pl.kernel,
        mesh=mesh,
        out_type=jax.ShapeDtypeStruct((n_rows, H // 2), jnp.float32),
        scratch_types=[
            pltpu.VMEM((chunk,), jnp.int32),
            pltpu.VMEM((chunk,), jnp.int32),
            pltpu.VMEM((chunk, H // 2), jnp.float32),
            pltpu.VMEM((chunk, H // 2), jnp.float32),
            pltpu.SemaphoreType.DMA,
            pltpu.SemaphoreType.DMA,
            pltpu.SemaphoreType.DMA,
            pltpu.SemaphoreType.DMA,
        ],
    )
    def gather(table_hbm, idx_hbm, out_hbm,
               ia, ib, ra, rb, sga, sgb, soa, sob):
        wid = _sc_wid()
        iv = (ia, ib)
        rv = (ra, rb)
        sg = (sga, sgb)
        so = (soa, sob)

        def start_gather(c):
            p = c % 2
            base = pl.multiple_of(wid * per_w + c * chunk, chunk)
            pltpu.sync_copy(idx_hbm.at[pl.ds(base, chunk)], iv[p])
            return pltpu.async_copy(table_hbm.at[iv[p]], rv[p], sg[p])

        pend_in = [start_gather(0), None]
        pend_out = [None, None]
        for c in range(n_chunks):
            p = c % 2
            q = (c + 1) % 2
            base = pl.multiple_of(wid * per_w + c * chunk, chunk)
            pend_in[p].wait()
            dout = pltpu.async_copy(rv[p], out_hbm.at[pl.ds(base, chunk)],
                                    so[p])
            if c + 1 < n_chunks:
                if pend_out[q] is not None:
                    pend_out[q].wait()
                    pend_out[q] = None
                pend_in[q] = start_gather(c + 1)
            pend_out[p] = dout
        for pr in pend_out:
            if pr is not None:
                pr.wait()

    return gather


@functools.cache
def _sc_kernels():
    mesh = plsc.VectorSubcoreMesh(core_axis_name="c", subcore_axis_name="s")
    return (
        _make_sc_dispatch(mesh, 16),
        _make_sc_gather(mesh, 2 * T, 16),
    )


_FBM = 1024
_FBF = 512


def _ffn_kernel(x_ref, w1_ref, b1_ref, w2_ref, b2_ref, o_ref, acc_ref):
    j = pl.program_id(1)
    nj = pl.num_programs(1)
    h = lax.dot_general(x_ref[...], w1_ref[...], (((1,), (1,)), ((), ())),
                        preferred_element_type=jnp.float32)
    h = h + b1_ref[...]
    g = 0.5 * h * (1.0 + lax.erf(h * 0.7071067811865476))
    contrib = lax.dot_general(g.astype(jnp.bfloat16), w2_ref[...],
                              (((1,), (1,)), ((), ())),
                              preferred_element_type=jnp.float32)

    @pl.when(j == 0)
    def _():
        acc_ref[...] = b2_ref[...] + contrib

    @pl.when(j > 0)
    def _():
        acc_ref[...] = acc_ref[...] + contrib

    @pl.when(j == nj - 1)
    def _():
        o_ref[...] = acc_ref[...].astype(jnp.bfloat16)


def _run_ffn(x, w1, b1, w2, b2):
    grid = (EC // _FBM, FF // _FBF)
    return pl.pallas_call(
        _ffn_kernel,
        grid=grid,
        in_specs=[
            pl.BlockSpec((_FBM, H), lambda i, j: (i, 0)),
            pl.BlockSpec((_FBF, H), lambda i, j: (j, 0)),
            pl.BlockSpec((1, _FBF), lambda i, j: (0, j)),
            pl.BlockSpec((H, _FBF), lambda i, j: (0, j)),
            pl.BlockSpec((1, H), lambda i, j: (0, 0)),
        ],
        out_specs=pl.BlockSpec((_FBM, H), lambda i, j: (i, 0)),
        out_shape=jax.ShapeDtypeStruct((EC, H), jnp.bfloat16),
        scratch_shapes=[pltpu.VMEM((_FBM, H), jnp.float32)],
    )(x, w1, b1, w2, b2)


_CBT = 512


def _combine_kernel(r0_ref, r1_ref, w0_ref, w1_ref, o_ref):
    w0 = w0_ref[...]
    w1 = w1_ref[...]
    r0 = r0_ref[...].astype(jnp.float32)
    r1 = r1_ref[...].astype(jnp.float32)
    z = jnp.zeros_like(r0)
    o_ref[...] = (jnp.where(w0 > 0.0, w0 * r0, z)
                  + jnp.where(w1 > 0.0, w1 * r1, z))


def _run_combine(r0, r1, w0, w1):
    grid = (T // _CBT,)
    return pl.pallas_call(
        _combine_kernel,
        grid=grid,
        in_specs=[
            pl.BlockSpec((_CBT, H), lambda i: (i, 0)),
            pl.BlockSpec((_CBT, H), lambda i: (i, 0)),
            pl.BlockSpec((_CBT, 1), lambda i: (i, 0)),
            pl.BlockSpec((_CBT, 1), lambda i: (i, 0)),
        ],
        out_specs=pl.BlockSpec((_CBT, H), lambda i: (i, 0)),
        out_shape=jax.ShapeDtypeStruct((T, H), jnp.float32),
    )(r0, r1, w0, w1)


def kernel(hidden_states, Wr, W1, b1, W2, b2):
    orig_shape = hidden_states.shape
    x = hidden_states.reshape(-1, H)

    p0, p1, i0, i1 = _run_router(x, Wr)
    ds0, ds1, d0, d1, w0, w1 = _run_meta(
        i0.reshape(64, 128), i1.reshape(64, 128),
        p0.reshape(64, 128), p1.reshape(64, 128))

    sc_dispatch, sc_gather_combine = _sc_kernels()

    def _pack(a):
        return lax.bitcast_convert_type(
            a.reshape(a.shape[0], H // 2, 2), jnp.float32)

    def _unpack(a):
        return lax.bitcast_convert_type(a, jnp.bfloat16).reshape(-1, H)

    x_pk = _pack(x.astype(jnp.bfloat16))
    permuted = _unpack(sc_dispatch(x_pk, ds0.reshape(-1), ds1.reshape(-1)))

    expert_out = _run_ffn(permuted, W1.astype(jnp.bfloat16),
                          b1.reshape(1, FF), W2.astype(jnp.bfloat16),
                          b2.reshape(1, H))

    didx = jnp.concatenate([d0.reshape(-1), d1.reshape(-1)])
    rows = _unpack(sc_gather_combine(_pack(expert_out), didx))

    out = _run_combine(rows[:T], rows[T:],
                       w0.reshape(T, 1), w1.reshape(T, 1))
    return out.reshape(orig_shape)


# speedup vs baseline: 1.4819x; 1.4819x over previous
import functools

import jax
import jax.numpy as jnp
from jax import lax
from jax.experimental import pallas as pl
from jax.experimental.pallas import tpu as pltpu
from jax.experimental.pallas import tpu_sc as plsc

E = 8
CAP = 512
EC = E * CAP
H = 2048
FF = 8192
T = 8192


_RBT = 512


def _router_kernel(x_ref, wr_ref, p0_ref, p1_ref, i0_ref, i1_ref):
    x = x_ref[...]
    wr = wr_ref[...]
    logits = lax.dot_general(x, wr, (((1,), (1,)), ((), ())),
                             preferred_element_type=jnp.float32)
    m = jnp.max(logits, axis=-1, keepdims=True)
    ex = jnp.exp(logits - m)
    probs = ex / jnp.sum(ex, axis=-1, keepdims=True)
    e_iota = lax.broadcasted_iota(jnp.int32, probs.shape, 1)
    big = jnp.int32(999)
    p0 = jnp.max(probs, axis=-1, keepdims=True)
    i0 = jnp.min(jnp.where(probs == p0, e_iota, big), axis=-1, keepdims=True)
    masked = jnp.where(e_iota == i0, jnp.float32(-1.0), probs)
    p1 = jnp.max(masked, axis=-1, keepdims=True)
    i1 = jnp.min(jnp.where(masked == p1, e_iota, big), axis=-1, keepdims=True)
    p0_ref[...] = p0
    p1_ref[...] = p1
    i0_ref[...] = i0
    i1_ref[...] = i1


def _run_router(x, wr):
    grid = (T // _RBT,)
    return pl.pallas_call(
        _router_kernel,
        grid=grid,
        in_specs=[
            pl.BlockSpec((_RBT, H), lambda i: (i, 0)),
            pl.BlockSpec((E, H), lambda i: (0, 0)),
        ],
        out_specs=[
            pl.BlockSpec((_RBT, 1), lambda i: (i, 0)),
            pl.BlockSpec((_RBT, 1), lambda i: (i, 0)),
            pl.BlockSpec((_RBT, 1), lambda i: (i, 0)),
            pl.BlockSpec((_RBT, 1), lambda i: (i, 0)),
        ],
        out_shape=[
            jax.ShapeDtypeStruct((T, 1), jnp.float32),
            jax.ShapeDtypeStruct((T, 1), jnp.float32),
            jax.ShapeDtypeStruct((T, 1), jnp.int32),
            jax.ShapeDtypeStruct((T, 1), jnp.int32),
        ],
    )(x, wr)


def _meta_kernel(i0_ref, i1_ref, p0_ref, p1_ref,
                 ds0_ref, ds1_ref, d0_ref, d1_ref, w0_ref, w1_ref):
    e0 = i0_ref[...]
    e1 = i1_ref[...]
    p0 = p0_ref[...]
    p1 = p1_ref[...]

    r_iota = lax.broadcasted_iota(jnp.int32, (128, 128), 0)
    c_iota = lax.broadcasted_iota(jnp.int32, (128, 128), 1)
    U128 = jnp.where(r_iota < c_iota, 1.0, 0.0).astype(jnp.float32)
    r64 = lax.broadcasted_iota(jnp.int32, (64, 64), 0)
    c64 = lax.broadcasted_iota(jnp.int32, (64, 64), 1)
    L64 = jnp.where(c64 < r64, 1.0, 0.0).astype(jnp.float32)
    ones_col = jnp.ones((128, 1), jnp.float32)

    rank0 = jnp.zeros((64, 128), jnp.float32)
    rank1 = jnp.zeros((64, 128), jnp.float32)
    for e in range(E):
        a = (e0 == e).astype(jnp.float32)
        b = (e1 == e).astype(jnp.float32)
        s = a + b
        pfx = lax.dot_general(s, U128, (((1,), (0,)), ((), ())),
                              preferred_element_type=jnp.float32)
        rs = lax.dot_general(s, ones_col, (((1,), (0,)), ((), ())),
                             preferred_element_type=jnp.float32)
        offs = lax.dot_general(L64, rs, (((1,), (0,)), ((), ())),
                               preferred_element_type=jnp.float32)
        excl = pfx + offs
        rank0 = rank0 + a * excl
        rank1 = rank1 + b * (excl + a)

    e0f = e0.astype(jnp.float32)
    e1f = e1.astype(jnp.float32)
    dest0 = e0f * float(CAP) + rank0
    dest1 = e1f * float(CAP) + rank1
    v0 = rank0 < float(CAP)
    v1 = rank1 < float(CAP)
    dest0i = dest0.astype(jnp.int32)
    dest1i = dest1.astype(jnp.int32)
    ds0_ref[...] = jnp.where(v0, dest0i, EC)
    ds1_ref[...] = jnp.where(v1, dest1i, EC)
    d0_ref[...] = jnp.where(v0, dest0i, 0)
    d1_ref[...] = jnp.where(v1, dest1i, 0)
    w0_ref[...] = p0 * v0.astype(jnp.float32)
    w1_ref[...] = p1 * v1.astype(jnp.float32)


def _run_meta(i0, i1, p0, p1):
    spec = pl.BlockSpec((64, 128), lambda: (0, 0))
    return pl.pallas_call(
        _meta_kernel,
        in_specs=[spec] * 4,
        out_specs=[spec] * 6,
        out_shape=[
            jax.ShapeDtypeStruct((64, 128), jnp.int32),
            jax.ShapeDtypeStruct((64, 128), jnp.int32),
            jax.ShapeDtypeStruct((64, 128), jnp.int32),
            jax.ShapeDtypeStruct((64, 128), jnp.int32),
            jax.ShapeDtypeStruct((64, 128), jnp.float32),
            jax.ShapeDtypeStruct((64, 128), jnp.float32),
        ],
    )(i0, i1, p0, p1)


PEC = EC + 8


def _sc_wid():
    return lax.axis_index("s") * 2 + lax.axis_index("c")


def _make_sc_dispatch(mesh, chunk):
    n_workers = 32
    per_w = T // n_workers
    n_chunks = per_w // chunk

    @functools.partial(
        pl.kernel,
        mesh=mesh,
        out_type=jax.ShapeDtypeStruct((PEC, H), jnp.float32),
        scratch_types=[
            pltpu.VMEM((chunk,), jnp.int32),
            pltpu.VMEM((chunk,), jnp.int32),
            pltpu.VMEM((chunk,), jnp.int32),
            pltpu.VMEM((chunk,), jnp.int32),
            pltpu.VMEM((chunk, H), jnp.float32),
            pltpu.VMEM((chunk, H), jnp.float32),
            pltpu.SemaphoreType.DMA,
            pltpu.SemaphoreType.DMA,
            pltpu.SemaphoreType.DMA,
            pltpu.SemaphoreType.DMA,
        ],
    )
    def dispatch(x_hbm, ds0_hbm, ds1_hbm, out_hbm,
                 i0a, i0b, i1a, i1b, ra, rb, sia, sib, soa, sob):
        wid = _sc_wid()
        i0v = (i0a, i0b)
        i1v = (i1a, i1b)
        rv = (ra, rb)
        si = (sia, sib)
        so = (soa, sob)

        def start_in(c):
            p = c % 2
            base = pl.multiple_of(wid * per_w + c * chunk, chunk)
            pltpu.sync_copy(ds0_hbm.at[pl.ds(base, chunk)], i0v[p])
            pltpu.sync_copy(ds1_hbm.at[pl.ds(base, chunk)], i1v[p])
            return pltpu.async_copy(x_hbm.at[pl.ds(base, chunk)], rv[p], si[p])

        pend_in = [start_in(0), None]
        pend_out = [None, None]
        for c in range(n_chunks):
            p = c % 2
            q = (c + 1) % 2
            pend_in[p].wait()
            d0 = pltpu.async_copy(rv[p], out_hbm.at[i0v[p]], so[p])
            d1 = pltpu.async_copy(rv[p], out_hbm.at[i1v[p]], so[p])
            if c + 1 < n_chunks:
                if pend_out[q] is not None:
                    for d in pend_out[q]:
                        d.wait()
                    pend_out[q] = None
                pend_in[q] = start_in(c + 1)
            pend_out[p] = (d0, d1)
        for pr in pend_out:
            if pr is not None:
                for d in pr:
                    d.wait()

    return dispatch


def _make_sc_gather(mesh, n_rows, chunk):
    n_workers = 32
    per_w = n_rows // n_workers
    n_chunks = per_w // chunk

    @functools.partial(
        pl.kernel,
        mesh=mesh,
        out_type=jax.ShapeDtypeStruct((n_rows, H), jnp.float32),
        scratch_types=[
            pltpu.VMEM((chunk,), jnp.int32),
            pltpu.VMEM((chunk,), jnp.int32),
            pltpu.VMEM((chunk, H), jnp.float32),
            pltpu.VMEM((chunk, H), jnp.float32),
            pltpu.SemaphoreType.DMA,
            pltpu.SemaphoreType.DMA,
            pltpu.SemaphoreType.DMA,
            pltpu.SemaphoreType.DMA,
        ],
    )
    def gather(table_hbm, idx_hbm, out_hbm,
               ia, ib, ra, rb, sga, sgb, soa, sob):
        wid = _sc_wid()
        iv = (ia, ib)
        rv = (ra, rb)
        sg = (sga, sgb)
        so = (soa, sob)

        def start_gather(c):
            p = c % 2
            base = pl.multiple_of(wid * per_w + c * chunk, chunk)
            pltpu.sync_copy(idx_hbm.at[pl.ds(base, chunk)], iv[p])
            return pltpu.async_copy(table_hbm.at[iv[p]], rv[p], sg[p])

        pend_in = [start_gather(0), None]
        pend_out = [None, None]
        for c in range(n_chunks):
            p = c % 2
            q = (c + 1) % 2
            base = pl.multiple_of(wid * per_w + c * chunk, chunk)
            pend_in[p].wait()
            dout = pltpu.async_copy(rv[p], out_hbm.at[pl.ds(base, chunk)],
                                    so[p])
            if c + 1 < n_chunks:
                if pend_out[q] is not None:
                    pend_out[q].wait()
                    pend_out[q] = None
                pend_in[q] = start_gather(c + 1)
            pend_out[p] = dout
        for pr in pend_out:
            if pr is not None:
                pr.wait()

    return gather


@functools.cache
def _sc_kernels():
    mesh = plsc.VectorSubcoreMesh(core_axis_name="c", subcore_axis_name="s")
    return (
        _make_sc_dispatch(mesh, 16),
        _make_sc_gather(mesh, 2 * T, 16),
    )


_FBM = 1024
_FBF = 512


def _ffn_kernel(x_ref, w1_ref, b1_ref, w2_ref, b2_ref, o_ref, acc_ref):
    j = pl.program_id(1)
    nj = pl.num_programs(1)
    h = lax.dot_general(x_ref[...].astype(jnp.bfloat16), w1_ref[...],
                        (((1,), (1,)), ((), ())),
                        preferred_element_type=jnp.float32)
    h = h + b1_ref[...]
    g = 0.5 * h * (1.0 + lax.erf(h * 0.7071067811865476))
    contrib = lax.dot_general(g.astype(jnp.bfloat16), w2_ref[...],
                              (((1,), (1,)), ((), ())),
                              preferred_element_type=jnp.float32)

    @pl.when(j == 0)
    def _():
        acc_ref[...] = b2_ref[...] + contrib

    @pl.when(j > 0)
    def _():
        acc_ref[...] = acc_ref[...] + contrib

    @pl.when(j == nj - 1)
    def _():
        o_ref[...] = acc_ref[...]


def _run_ffn(x, w1, b1, w2, b2):
    grid = (EC // _FBM, FF // _FBF)
    return pl.pallas_call(
        _ffn_kernel,
        grid=grid,
        in_specs=[
            pl.BlockSpec((_FBM, H), lambda i, j: (i, 0)),
            pl.BlockSpec((_FBF, H), lambda i, j: (j, 0)),
            pl.BlockSpec((1, _FBF), lambda i, j: (0, j)),
            pl.BlockSpec((H, _FBF), lambda i, j: (0, j)),
            pl.BlockSpec((1, H), lambda i, j: (0, 0)),
        ],
        out_specs=pl.BlockSpec((_FBM, H), lambda i, j: (i, 0)),
        out_shape=jax.ShapeDtypeStruct((EC, H), jnp.float32),
        scratch_shapes=[pltpu.VMEM((_FBM, H), jnp.float32)],
    )(x, w1, b1, w2, b2)


_CBT = 512


def _combine_kernel(r0_ref, r1_ref, w0_ref, w1_ref, o_ref):
    w0 = w0_ref[...]
    w1 = w1_ref[...]
    r0 = r0_ref[...].astype(jnp.float32)
    r1 = r1_ref[...].astype(jnp.float32)
    z = jnp.zeros_like(r0)
    o_ref[...] = (jnp.where(w0 > 0.0, w0 * r0, z)
                  + jnp.where(w1 > 0.0, w1 * r1, z))


def _run_combine(r0, r1, w0, w1):
    grid = (T // _CBT,)
    return pl.pallas_call(
        _combine_kernel,
        grid=grid,
        in_specs=[
            pl.BlockSpec((_CBT, H), lambda i: (i, 0)),
            pl.BlockSpec((_CBT, H), lambda i: (i, 0)),
            pl.BlockSpec((_CBT, 1), lambda i: (i, 0)),
            pl.BlockSpec((_CBT, 1), lambda i: (i, 0)),
        ],
        out_specs=pl.BlockSpec((_CBT, H), lambda i: (i, 0)),
        out_shape=jax.ShapeDtypeStruct((T, H), jnp.float32),
    )(r0, r1, w0, w1)


def kernel(hidden_states, Wr, W1, b1, W2, b2):
    orig_shape = hidden_states.shape
    x = hidden_states.reshape(-1, H)

    p0, p1, i0, i1 = _run_router(x, Wr)
    ds0, ds1, d0, d1, w0, w1 = _run_meta(
        i0.reshape(64, 128), i1.reshape(64, 128),
        p0.reshape(64, 128), p1.reshape(64, 128))

    sc_dispatch, sc_gather_combine = _sc_kernels()

    permuted = sc_dispatch(x, ds0.reshape(-1), ds1.reshape(-1))

    expert_out = _run_ffn(permuted, W1.astype(jnp.bfloat16),
                          b1.reshape(1, FF), W2.astype(jnp.bfloat16),
                          b2.reshape(1, H))

    didx = jnp.concatenate([d0.reshape(-1), d1.reshape(-1)])
    rows = sc_gather_combine(expert_out, didx)

    out = _run_combine(rows[:T], rows[T:],
                       w0.reshape(T, 1), w1.reshape(T, 1))
    return out.reshape(orig_shape)
